# baseline (speedup 1.0000x reference)
import functools

import jax
import jax.numpy as jnp
import numpy as np
from jax import lax
from jax.experimental import pallas as pl
from jax.experimental.pallas import tpu as pltpu
from jax.experimental.pallas import tpu_sc as plsc

TOTAL = 2048
SHUFFLE_NUM = 1024
ROWS = 4 * 4096
SC_ROWS = 9216
TC_ROWS = ROWS - SC_ROWS
TC_BLOCK = 256
NUM_WORKERS = 32
ROWS_PER_WORKER = SC_ROWS // NUM_WORKERS
CHUNK_ROWS = 8
CHUNKS = ROWS_PER_WORKER // CHUNK_ROWS
NBUF = 3
MAIN = (CHUNKS // NBUF) * NBUF
LANES = 16
GROUPS = TOTAL // LANES


def _perm_index():
    with jax.default_device(jax.local_devices(backend="cpu")[0]):
        pkey = jax.random.key(42)
        perm = jax.random.permutation(pkey, TOTAL)
        random_sort = perm[:SHUFFLE_NUM]
        random_index = jnp.sort(random_sort)
        index = jnp.arange(TOTAL, dtype=jnp.int32)
        index = index.at[random_index].set(random_sort.astype(jnp.int32))
        return np.asarray(index)


_IDX_NP = _perm_index()
_P_NP = np.zeros((TOTAL, TOTAL), np.float32)
_P_NP[_IDX_NP, np.arange(TOTAL)] = 1.0


def _sc_body(x_hbm, idx_hbm, out_hbm, idx_v, *refs):
    ins = refs[0:NBUF]
    outs = refs[NBUF:2 * NBUF]
    isems = refs[2 * NBUF:3 * NBUF]
    osems = refs[3 * NBUF:4 * NBUF]
    wid = lax.axis_index("c") * 16 + lax.axis_index("s")
    pltpu.sync_copy(idx_hbm, idx_v)
    row0 = wid * ROWS_PER_WORKER

    def in_copy(c, b):
        return pltpu.make_async_copy(
            x_hbm.at[pl.ds(row0 + c * CHUNK_ROWS, CHUNK_ROWS)], ins[b], isems[b])

    def out_copy(c, b):
        return pltpu.make_async_copy(
            outs[b], out_hbm.at[pl.ds(row0 + c * CHUNK_ROWS, CHUNK_ROWS)], osems[b])

    def compute(b):
        in_v, out_v = ins[b], outs[b]

        @plsc.parallel_loop(0, GROUPS, unroll=8)
        def _(g):
            col = g * LANES
            iv = idx_v[pl.ds(col, LANES)]
            for r in range(CHUNK_ROWS):
                rv = jnp.full((LANES,), r, jnp.int32)
                vals = plsc.load_gather(in_v, [rv, iv])
                out_v[r, pl.ds(col, LANES)] = vals

    for c in range(NBUF - 1):
        in_copy(c, c).start()

    def ring_body(p, carry):
        c0 = p * NBUF
        for b in range(NBUF):
            c = c0 + b

            @pl.when(c + NBUF - 1 < CHUNKS)
            def _():
                in_copy(c + NBUF - 1, (b + NBUF - 1) % NBUF).start()

            in_copy(c, b).wait()

            @pl.when(c >= NBUF)
            def _():
                out_copy(c - NBUF, b).wait()

            compute(b)
            out_copy(c, b).start()
        return carry

    lax.fori_loop(0, MAIN // NBUF, ring_body, 0)
    for c in range(MAIN, CHUNKS):
        b = c % NBUF
        in_copy(c, b).wait()
        out_copy(c - NBUF, b).wait()
        compute(b)
        out_copy(c, b).start()
    for c in range(CHUNKS - NBUF, CHUNKS):
        out_copy(c, c % NBUF).wait()


def _tc_body(x_ref, p_ref, out_ref):
    xb = x_ref[...].astype(jnp.bfloat16)
    out_ref[...] = jnp.dot(xb, p_ref[...],
                           preferred_element_type=jnp.float32)


@jax.jit
def _shuffle(x2, idx, p_bf16):
    mesh = plsc.VectorSubcoreMesh(core_axis_name="c", subcore_axis_name="s")
    sc_k = functools.partial(
        pl.kernel,
        mesh=mesh,
        out_type=jax.ShapeDtypeStruct((SC_ROWS, TOTAL), jnp.float32),
        scratch_types=(
            [pltpu.VMEM((TOTAL,), jnp.int32)]
            + [pltpu.VMEM((CHUNK_ROWS, TOTAL), jnp.float32)] * (2 * NBUF)
            + [pltpu.SemaphoreType.DMA] * (2 * NBUF)
        ),
        compiler_params=pltpu.CompilerParams(needs_layout_passes=False),
    )(_sc_body)
    out_sc = sc_k(x2, idx)

    n_tc_blocks = TC_ROWS // TC_BLOCK
    out_tc = pl.pallas_call(
        _tc_body,
        grid=(n_tc_blocks,),
        in_specs=[
            pl.BlockSpec((TC_BLOCK, TOTAL),
                         lambda i: (SC_ROWS // TC_BLOCK + i, 0)),
            pl.BlockSpec((TOTAL, TOTAL), lambda i: (0, 0)),
        ],
        out_specs=pl.BlockSpec((TC_BLOCK, TOTAL), lambda i: (i, 0)),
        out_shape=jax.ShapeDtypeStruct((TC_ROWS, TOTAL), jnp.float32),
    )(x2, p_bf16)

    return jnp.concatenate([out_sc, out_tc], axis=0)


def kernel(input):
    idx = jnp.asarray(_IDX_NP)
    p_bf16 = jnp.asarray(_P_NP, dtype=jnp.bfloat16)
    out2 = _shuffle(input.reshape(ROWS, TOTAL), idx, p_bf16)
    return out2.reshape(input.shape)


# speedup vs baseline: 1.7632x; 1.7632x over previous
import functools

import jax
import jax.numpy as jnp
import numpy as np
from jax import lax
from jax.experimental import pallas as pl
from jax.experimental.pallas import tpu as pltpu
from jax.experimental.pallas import tpu_sc as plsc

TOTAL = 2048
SHUFFLE_NUM = 1024
ROWS = 4 * 4096
NUM_WORKERS = 32
ROWS_PER_WORKER = ROWS // NUM_WORKERS
CHUNK_ROWS = 8
CHUNKS = ROWS_PER_WORKER // CHUNK_ROWS
NBUF = 3
MAIN = (CHUNKS // NBUF) * NBUF
REM = CHUNKS - MAIN
LANES = 16
GROUPS = TOTAL // LANES


def _perm_index():
    with jax.default_device(jax.local_devices(backend="cpu")[0]):
        pkey = jax.random.key(42)
        perm = jax.random.permutation(pkey, TOTAL)
        random_sort = perm[:SHUFFLE_NUM]
        random_index = jnp.sort(random_sort)
        index = jnp.arange(TOTAL, dtype=jnp.int32)
        index = index.at[random_index].set(random_sort.astype(jnp.int32))
        return np.asarray(index)


_IDX_NP = _perm_index()


def _body(x_hbm, idx_hbm, out_hbm, idx_v, *refs):
    ins = refs[0:NBUF]
    outs = refs[NBUF:2 * NBUF]
    isems = refs[2 * NBUF:3 * NBUF]
    osems = refs[3 * NBUF:4 * NBUF]
    wid = lax.axis_index("c") * 16 + lax.axis_index("s")
    row0 = wid * ROWS_PER_WORKER

    def in_copy(c, b):
        return pltpu.make_async_copy(
            x_hbm.at[pl.ds(row0 + c * CHUNK_ROWS, CHUNK_ROWS)], ins[b], isems[b])

    def out_copy(c, b):
        return pltpu.make_async_copy(
            outs[b], out_hbm.at[pl.ds(row0 + c * CHUNK_ROWS, CHUNK_ROWS)], osems[b])

    def compute(b):
        in_v, out_v = ins[b], outs[b]

        @plsc.parallel_loop(0, GROUPS, unroll=8)
        def _(g):
            col = g * LANES
            iv = idx_v[pl.ds(col, LANES)]
            for r in range(CHUNK_ROWS):
                rv = jnp.full((LANES,), r, jnp.int32)
                vals = plsc.load_gather(in_v, [rv, iv])
                out_v[r, pl.ds(col, LANES)] = vals

    for c in range(NBUF - 1):
        in_copy(c, c).start()
    pltpu.sync_copy(idx_hbm, idx_v)

    def ring_body(p, carry):
        c0 = p * NBUF
        for b in range(NBUF):
            c = c0 + b

            @pl.when(c + NBUF - 1 < CHUNKS)
            def _():
                in_copy(c + NBUF - 1, (b + NBUF - 1) % NBUF).start()

            in_copy(c, b).wait()

            @pl.when(c >= NBUF)
            def _():
                out_copy(c - NBUF, b).wait()

            compute(b)
            out_copy(c, b).start()
        return carry

    lax.fori_loop(0, MAIN // NBUF, ring_body, 0)
    for c in range(MAIN, CHUNKS):
        b = c % NBUF
        in_copy(c, b).wait()
        out_copy(c - NBUF, b).wait()
        compute(b)
        out_copy(c, b).start()
    for c in range(CHUNKS - NBUF, CHUNKS):
        out_copy(c, c % NBUF).wait()


@jax.jit
def _shuffle(x2, idx):
    mesh = plsc.VectorSubcoreMesh(core_axis_name="c", subcore_axis_name="s")
    k = functools.partial(
        pl.kernel,
        mesh=mesh,
        out_type=jax.ShapeDtypeStruct((ROWS, TOTAL), jnp.float32),
        scratch_types=(
            [pltpu.VMEM((TOTAL,), jnp.int32)]
            + [pltpu.VMEM((CHUNK_ROWS, TOTAL), jnp.float32)] * (2 * NBUF)
            + [pltpu.SemaphoreType.DMA] * (2 * NBUF)
        ),
        compiler_params=pltpu.CompilerParams(needs_layout_passes=False),
    )(_body)
    return k(x2, idx)


def kernel(input):
    idx = jnp.asarray(_IDX_NP)
    out2 = _shuffle(input.reshape(ROWS, TOTAL), idx)
    return out2.reshape(input.shape)
